# baseline (device time: 7556 ns/iter reference)
import jax
import jax.numpy as jnp
from jax import lax
from jax.experimental import pallas as pl
from jax.experimental.pallas import tpu as pltpu

N_DEV = 16


def kernel(x):
    m, n = x.shape

    def body(x_hbm, out_hbm, xv, yv, halo_ref, send_sem, recv_sem, dsem):
        my = lax.axis_index("i")
        has_left = my > 0
        has_right = my < N_DEV - 1

        barrier = pltpu.get_barrier_semaphore()

        @pl.when(has_left)
        def _():
            pl.semaphore_signal(
                barrier, inc=1, device_id=(my - 1,),
                device_id_type=pl.DeviceIdType.MESH,
            )

        @pl.when(has_right)
        def _():
            pl.semaphore_signal(
                barrier, inc=1, device_id=(my + 1,),
                device_id_type=pl.DeviceIdType.MESH,
            )

        copy_in = pltpu.make_async_copy(x_hbm, xv, dsem.at[0])
        copy_in.start()

        n_nbrs = has_left.astype(jnp.int32) + has_right.astype(jnp.int32)
        pl.semaphore_wait(barrier, n_nbrs)

        send_l = pltpu.make_async_remote_copy(
            src_ref=x_hbm.at[pl.ds(0, 1)],
            dst_ref=halo_ref.at[pl.ds(1, 1)],
            send_sem=send_sem.at[0],
            recv_sem=recv_sem.at[1],
            device_id=(my - 1,),
            device_id_type=pl.DeviceIdType.MESH,
        )
        send_r = pltpu.make_async_remote_copy(
            src_ref=x_hbm.at[pl.ds(m - 1, 1)],
            dst_ref=halo_ref.at[pl.ds(0, 1)],
            send_sem=send_sem.at[1],
            recv_sem=recv_sem.at[0],
            device_id=(my + 1,),
            device_id_type=pl.DeviceIdType.MESH,
        )

        @pl.when(has_left)
        def _():
            send_l.start()

        @pl.when(has_right)
        def _():
            send_r.start()

        copy_in.wait()

        xvv = xv[:]
        yv[pl.ds(1, m - 2)] = (
            0.25 * (xvv[: m - 2] + xvv[2:]) + 0.5 * xvv[1 : m - 1]
        )

        copy_out = pltpu.make_async_copy(
            yv.at[pl.ds(8, m - 16)], out_hbm.at[pl.ds(8, m - 16)], dsem.at[1]
        )
        copy_out.start()

        @pl.when(has_left)
        def _():
            pltpu.make_async_remote_copy(
                src_ref=x_hbm.at[pl.ds(0, 1)],
                dst_ref=halo_ref.at[pl.ds(0, 1)],
                send_sem=send_sem.at[0],
                recv_sem=recv_sem.at[0],
                device_id=(my - 1,),
                device_id_type=pl.DeviceIdType.MESH,
            ).wait_recv()

        @pl.when(has_right)
        def _():
            pltpu.make_async_remote_copy(
                src_ref=x_hbm.at[pl.ds(0, 1)],
                dst_ref=halo_ref.at[pl.ds(1, 1)],
                send_sem=send_sem.at[1],
                recv_sem=recv_sem.at[1],
                device_id=(my + 1,),
                device_id_type=pl.DeviceIdType.MESH,
            ).wait_recv()

        top = halo_ref[pl.ds(0, 1)]
        bot = halo_ref[pl.ds(1, 1)]
        yv[pl.ds(0, 1)] = jnp.where(
            has_left, 0.25 * top + 0.5 * xvv[0:1] + 0.25 * xvv[1:2], xvv[0:1]
        )
        yv[pl.ds(m - 1, 1)] = jnp.where(
            has_right,
            0.25 * xvv[m - 2 : m - 1] + 0.5 * xvv[m - 1 : m] + 0.25 * bot,
            xvv[m - 1 : m],
        )
        row0 = pltpu.make_async_copy(
            yv.at[pl.ds(0, 8)], out_hbm.at[pl.ds(0, 8)], dsem.at[2]
        )
        rowl = pltpu.make_async_copy(
            yv.at[pl.ds(m - 8, 8)], out_hbm.at[pl.ds(m - 8, 8)], dsem.at[3]
        )
        row0.start()
        rowl.start()

        copy_out.wait()
        row0.wait()
        rowl.wait()

        @pl.when(has_left)
        def _():
            send_l.wait_send()

        @pl.when(has_right)
        def _():
            send_r.wait_send()

    return pl.pallas_call(
        body,
        out_shape=jax.ShapeDtypeStruct((m, n), x.dtype),
        in_specs=[pl.BlockSpec(memory_space=pl.ANY)],
        out_specs=pl.BlockSpec(memory_space=pl.ANY),
        scratch_shapes=[
            pltpu.VMEM((m, n), x.dtype),
            pltpu.VMEM((m, n), x.dtype),
            pltpu.VMEM((2, n), x.dtype),
            pltpu.SemaphoreType.DMA((2,)),
            pltpu.SemaphoreType.DMA((2,)),
            pltpu.SemaphoreType.DMA((4,)),
        ],
        compiler_params=pltpu.CompilerParams(collective_id=0),
    )(x)


# device time: 6695 ns/iter; 1.1286x vs baseline; 1.1286x over previous
import jax
import jax.numpy as jnp
from jax import lax
from jax.experimental import pallas as pl
from jax.experimental.pallas import tpu as pltpu

N_DEV = 16


def kernel(x):
    m, n = x.shape

    def body(x_hbm, out_hbm, xv, yv, halo_ref, send_sem, recv_sem, dsem, ready):
        my = lax.axis_index("i")
        has_left = my > 0
        has_right = my < N_DEV - 1

        barrier = pltpu.get_barrier_semaphore()
        pl.semaphore_signal(barrier, inc=1)
        pl.semaphore_wait(barrier, 1)

        copy_in = pltpu.make_async_copy(x_hbm, xv, dsem.at[0])
        copy_in.start()

        @pl.when(has_left)
        def _():
            pl.semaphore_signal(
                ready.at[1], inc=1, device_id=(my - 1,),
                device_id_type=pl.DeviceIdType.MESH,
            )

        @pl.when(has_right)
        def _():
            pl.semaphore_signal(
                ready.at[0], inc=1, device_id=(my + 1,),
                device_id_type=pl.DeviceIdType.MESH,
            )

        send_l = pltpu.make_async_remote_copy(
            src_ref=x_hbm.at[pl.ds(0, 1)],
            dst_ref=halo_ref.at[pl.ds(1, 1)],
            send_sem=send_sem.at[0],
            recv_sem=recv_sem.at[1],
            device_id=(my - 1,),
            device_id_type=pl.DeviceIdType.MESH,
        )
        send_r = pltpu.make_async_remote_copy(
            src_ref=x_hbm.at[pl.ds(m - 1, 1)],
            dst_ref=halo_ref.at[pl.ds(0, 1)],
            send_sem=send_sem.at[1],
            recv_sem=recv_sem.at[0],
            device_id=(my + 1,),
            device_id_type=pl.DeviceIdType.MESH,
        )

        @pl.when(has_left)
        def _():
            pl.semaphore_wait(ready.at[0], 1)
            send_l.start()

        @pl.when(has_right)
        def _():
            pl.semaphore_wait(ready.at[1], 1)
            send_r.start()

        copy_in.wait()
        xvv = xv[:]
        yv[pl.ds(1, m - 2)] = (
            0.25 * (xvv[: m - 2] + xvv[2:]) + 0.5 * xvv[1 : m - 1]
        )

        copy_out = pltpu.make_async_copy(
            yv.at[pl.ds(8, m - 16)], out_hbm.at[pl.ds(8, m - 16)], dsem.at[1]
        )
        copy_out.start()

        @pl.when(has_left)
        def _():
            pltpu.make_async_remote_copy(
                src_ref=halo_ref.at[pl.ds(0, 1)],
                dst_ref=halo_ref.at[pl.ds(0, 1)],
                send_sem=send_sem.at[0],
                recv_sem=recv_sem.at[0],
                device_id=(my - 1,),
                device_id_type=pl.DeviceIdType.MESH,
            ).wait_recv()

        @pl.when(has_right)
        def _():
            pltpu.make_async_remote_copy(
                src_ref=halo_ref.at[pl.ds(1, 1)],
                dst_ref=halo_ref.at[pl.ds(1, 1)],
                send_sem=send_sem.at[1],
                recv_sem=recv_sem.at[1],
                device_id=(my + 1,),
                device_id_type=pl.DeviceIdType.MESH,
            ).wait_recv()

        top = halo_ref[pl.ds(0, 1)]
        bot = halo_ref[pl.ds(1, 1)]
        yv[pl.ds(0, 1)] = jnp.where(
            has_left, 0.25 * top + 0.5 * xvv[0:1] + 0.25 * xvv[1:2], xvv[0:1]
        )
        yv[pl.ds(m - 1, 1)] = jnp.where(
            has_right,
            0.25 * xvv[m - 2 : m - 1] + 0.5 * xvv[m - 1 : m] + 0.25 * bot,
            xvv[m - 1 : m],
        )
        row0 = pltpu.make_async_copy(
            yv.at[pl.ds(0, 8)], out_hbm.at[pl.ds(0, 8)], dsem.at[2]
        )
        rowl = pltpu.make_async_copy(
            yv.at[pl.ds(m - 8, 8)], out_hbm.at[pl.ds(m - 8, 8)], dsem.at[3]
        )
        row0.start()
        rowl.start()

        copy_out.wait()
        row0.wait()
        rowl.wait()

        @pl.when(has_left)
        def _():
            send_l.wait_send()

        @pl.when(has_right)
        def _():
            send_r.wait_send()

    return pl.pallas_call(
        body,
        out_shape=jax.ShapeDtypeStruct((m, n), x.dtype),
        in_specs=[pl.BlockSpec(memory_space=pl.ANY)],
        out_specs=pl.BlockSpec(memory_space=pl.ANY),
        scratch_shapes=[
            pltpu.VMEM((m, n), x.dtype),
            pltpu.VMEM((m, n), x.dtype),
            pltpu.VMEM((2, n), x.dtype),
            pltpu.SemaphoreType.DMA((2,)),
            pltpu.SemaphoreType.DMA((2,)),
            pltpu.SemaphoreType.DMA((4,)),
            pltpu.SemaphoreType.REGULAR((2,)),
        ],
        compiler_params=pltpu.CompilerParams(collective_id=0),
    )(x)
